# baseline (speedup 1.0000x reference)
import jax
import jax.numpy as jnp
from jax import lax
from jax.experimental import pallas as pl
from jax.experimental.pallas import tpu as pltpu
from jax.experimental.pallas import tpu_sc as plsc

NUM_CORES = 2
NUM_SUBCORES = 16
NW = NUM_CORES * NUM_SUBCORES
LANES = 16

BATCH = 16384
K = 32
NROWS = 1000000
B_PER_W = BATCH // NW
NQ = B_PER_W // 128


def _sc_kernel(uidx_hbm, iidx_hbm, uemb_hbm, iemb_hbm, out_hbm,
               uidx_v, iidx_v, u_t, v_t, out_v, sem):
    wid = lax.axis_index("s") * NUM_CORES + lax.axis_index("c")

    pltpu.sync_copy(uidx_hbm.at[wid], uidx_v)
    pltpu.sync_copy(iidx_hbm.at[wid], iidx_v)

    copies = []
    for q in range(NQ):
        cols = pl.ds(q * 128, 128)
        for k in range(K):
            copies.append(pltpu.async_copy(
                uemb_hbm.at[k].at[uidx_v.at[q]], u_t.at[k, cols], sem))
            copies.append(pltpu.async_copy(
                iemb_hbm.at[k].at[iidx_v.at[q]], v_t.at[k, cols], sem))
    for c in copies:
        c.wait()

    zero = jnp.zeros((LANES,), jnp.float32)

    @pl.loop(0, B_PER_W // LANES)
    def _(c):
        cols = pl.ds(c * LANES, LANES)
        acc = zero
        for k in range(K):
            u = jnp.maximum(u_t[k, cols], zero)
            v = jnp.maximum(v_t[k, cols], zero)
            acc = acc + u * v
        out_v[cols] = acc

    pltpu.sync_copy(out_v, out_hbm.at[wid])


@jax.jit
def kernel(user_idx, item_idx, user_emb, item_emb):
    uidx = user_idx.reshape(NW, NQ, 128)
    iidx = item_idx.reshape(NW, NQ, 128)
    mesh = plsc.VectorSubcoreMesh(core_axis_name="c", subcore_axis_name="s")
    cp = pltpu.CompilerParams(needs_layout_passes=False,
                              use_tc_tiling_on_sc=False)
    run = pl.kernel(
        _sc_kernel,
        out_type=jax.ShapeDtypeStruct((NW, B_PER_W), jnp.float32),
        mesh=mesh,
        scratch_types=[
            pltpu.VMEM((NQ, 128), jnp.int32),
            pltpu.VMEM((NQ, 128), jnp.int32),
            pltpu.VMEM((K, B_PER_W), jnp.float32),
            pltpu.VMEM((K, B_PER_W), jnp.float32),
            pltpu.VMEM((B_PER_W,), jnp.float32),
            pltpu.SemaphoreType.DMA,
        ],
        compiler_params=cp,
    )
    out = run(uidx, iidx, user_emb.T, item_emb.T)
    return out.reshape(BATCH)


# speedup vs baseline: 5.7049x; 5.7049x over previous
import jax
import jax.numpy as jnp
from jax import lax
from jax.experimental import pallas as pl
from jax.experimental.pallas import tpu as pltpu
from jax.experimental.pallas import tpu_sc as plsc

NUM_CORES = 2
NUM_SUBCORES = 16
NW = NUM_CORES * NUM_SUBCORES
LANES = 16

BATCH = 16384
K = 32
B_PER_W = BATCH // NW
IDX_CHUNK = 128
N_CHUNKS = B_PER_W // IDX_CHUNK


def _sc_kernel(uidx_hbm, iidx_hbm, uemb_hbm, iemb_hbm, out_hbm,
               uidx_v, iidx_v, urows_v, irows_v, out_v, sem):
    wid = lax.axis_index("s") * NUM_CORES + lax.axis_index("c")
    base = wid * B_PER_W

    pltpu.sync_copy(uidx_hbm.at[wid], uidx_v)
    pltpu.sync_copy(iidx_hbm.at[wid], iidx_v)

    copies = []
    for j in range(N_CHUNKS):
        copies.append(pltpu.async_copy(
            uemb_hbm.at[uidx_v.at[j]],
            urows_v.at[pl.ds(j * IDX_CHUNK, IDX_CHUNK)], sem))
        copies.append(pltpu.async_copy(
            iemb_hbm.at[iidx_v.at[j]],
            irows_v.at[pl.ds(j * IDX_CHUNK, IDX_CHUNK)], sem))
    for c in copies:
        c.wait()

    zero = jnp.zeros((LANES,), jnp.float32)
    lane = lax.iota(jnp.int32, LANES)
    last_lane = lane == (LANES - 1)

    @pl.loop(0, B_PER_W)
    def _(r):
        u0 = jnp.maximum(urows_v[r, pl.ds(0, LANES)], zero)
        u1 = jnp.maximum(urows_v[r, pl.ds(LANES, LANES)], zero)
        v0 = jnp.maximum(irows_v[r, pl.ds(0, LANES)], zero)
        v1 = jnp.maximum(irows_v[r, pl.ds(LANES, LANES)], zero)
        c = plsc.cumsum(u0 * v0 + u1 * v1)
        plsc.store_scatter(out_v, [jnp.zeros((LANES,), jnp.int32) + r], c,
                           mask=last_lane)

    pltpu.sync_copy(out_v, out_hbm.at[pl.ds(base, B_PER_W)])


@jax.jit
def kernel(user_idx, item_idx, user_emb, item_emb):
    uidx = user_idx.reshape(NW, N_CHUNKS, IDX_CHUNK)
    iidx = item_idx.reshape(NW, N_CHUNKS, IDX_CHUNK)
    mesh = plsc.VectorSubcoreMesh(core_axis_name="c", subcore_axis_name="s")
    cp = pltpu.CompilerParams(needs_layout_passes=False,
                              use_tc_tiling_on_sc=False)
    run = pl.kernel(
        _sc_kernel,
        out_type=jax.ShapeDtypeStruct((BATCH,), jnp.float32),
        mesh=mesh,
        scratch_types=[
            pltpu.VMEM((N_CHUNKS, IDX_CHUNK), jnp.int32),
            pltpu.VMEM((N_CHUNKS, IDX_CHUNK), jnp.int32),
            pltpu.VMEM((B_PER_W, K), jnp.float32),
            pltpu.VMEM((B_PER_W, K), jnp.float32),
            pltpu.VMEM((B_PER_W,), jnp.float32),
            pltpu.SemaphoreType.DMA,
        ],
        compiler_params=cp,
    )
    return run(uidx, iidx, user_emb, item_emb)


# speedup vs baseline: 11.0493x; 1.9368x over previous
import jax
import jax.numpy as jnp
from jax import lax
from jax.experimental import pallas as pl
from jax.experimental.pallas import tpu as pltpu
from jax.experimental.pallas import tpu_sc as plsc

NUM_CORES = 2
NUM_SUBCORES = 16
NW = NUM_CORES * NUM_SUBCORES
LANES = 16

BATCH = 16384
K = 32
NROWS = 1000000
B_PER_W = BATCH // NW
NQ = B_PER_W // 128

MAIN = (NROWS // 128) * 128
T_PER_W = (MAIN // 128) // 4
T_BLK = 7
N_IT = T_PER_W // T_BLK
TAIL = NROWS - MAIN


def _untile_kernel(uemb_hbm, iemb_hbm, utail_hbm, itail_hbm,
                   uflat_hbm, iflat_hbm, buf_v, tail_v, sem):
    wid = lax.axis_index("s") * NUM_CORES + lax.axis_index("c")
    t = wid // 16
    r = wid % 16
    g = r // 4
    qd = r % 4
    soff = pl.multiple_of(8 * g, 8)

    def move(emb, flat):
        @pl.loop(0, N_IT)
        def _(it):
            tile0 = qd * T_PER_W + it * T_BLK
            ins = []
            for c in range(T_BLK):
                la = pl.multiple_of((tile0 + c) * 128, 128)
                ins.append(pltpu.async_copy(
                    emb.at[pl.ds(soff, 8), pl.ds(la, 128)],
                    buf_v.at[c], sem))
            for i in ins:
                i.wait()
            outs = []
            for c in range(T_BLK):
                for kr in range(8):
                    doff = pl.multiple_of(
                        (8 * g + kr) * NROWS + (tile0 + c) * 128, 8)
                    outs.append(pltpu.async_copy(
                        buf_v.at[c, kr],
                        flat.at[pl.ds(doff, 128)], sem))
            for o in outs:
                o.wait()

    @pl.when(t == 0)
    def _():
        move(uemb_hbm, uflat_hbm)

    @pl.when(t == 1)
    def _():
        move(iemb_hbm, iflat_hbm)

    def move_tail(tail_hbm, flat):
        pltpu.sync_copy(tail_hbm, tail_v)
        for k in range(K):
            pltpu.async_copy(tail_v.at[k],
                             flat.at[pl.ds(k * NROWS + MAIN, TAIL)],
                             sem).wait()

    @pl.when(wid == 0)
    def _():
        move_tail(utail_hbm, uflat_hbm)

    @pl.when(wid == 16)
    def _():
        move_tail(itail_hbm, iflat_hbm)


def _gather_kernel(uidx_hbm, iidx_hbm, uflat_hbm, iflat_hbm, out_hbm,
                   uidx_v, iidx_v, gidx_u, gidx_i, u_t, v_t, out_v, sem):
    wid = lax.axis_index("s") * NUM_CORES + lax.axis_index("c")

    pltpu.sync_copy(uidx_hbm.at[wid], uidx_v)
    pltpu.sync_copy(iidx_hbm.at[wid], iidx_v)

    @pl.loop(0, NQ)
    def _(q):
        @pl.loop(0, 128 // LANES)
        def _(j):
            ds = pl.ds(j * LANES, LANES)
            ivu = uidx_v[q, ds]
            ivi = iidx_v[q, ds]
            for k in range(K):
                gidx_u[k, q, ds] = ivu + k * NROWS
                gidx_i[k, q, ds] = ivi + k * NROWS

    @pl.loop(0, NQ)
    def _(q):
        cols = pl.ds(q * 128, 128)
        for k in range(K):
            pltpu.async_copy(uflat_hbm.at[gidx_u.at[k, q]],
                             u_t.at[k, cols], sem)
            pltpu.async_copy(iflat_hbm.at[gidx_i.at[k, q]],
                             v_t.at[k, cols], sem)

    @pl.loop(0, NQ)
    def _(q):
        cols = pl.ds(q * 128, 128)
        for k in range(K):
            pltpu.make_async_copy(uflat_hbm.at[gidx_u.at[k, q]],
                                  u_t.at[k, cols], sem).wait()
            pltpu.make_async_copy(iflat_hbm.at[gidx_i.at[k, q]],
                                  v_t.at[k, cols], sem).wait()

    zero = jnp.zeros((LANES,), jnp.float32)

    @pl.loop(0, B_PER_W // LANES)
    def _(c):
        cols = pl.ds(c * LANES, LANES)
        acc = zero
        for k in range(K):
            u = jnp.maximum(u_t[k, cols], zero)
            v = jnp.maximum(v_t[k, cols], zero)
            acc = acc + u * v
        out_v[cols] = acc

    pltpu.sync_copy(out_v, out_hbm.at[wid])


@jax.jit
def kernel(user_idx, item_idx, user_emb, item_emb):
    uidx = user_idx.reshape(NW, NQ, 128)
    iidx = item_idx.reshape(NW, NQ, 128)
    mesh = plsc.VectorSubcoreMesh(core_axis_name="c", subcore_axis_name="s")
    cpt = pltpu.CompilerParams(needs_layout_passes=False,
                               use_tc_tiling_on_sc=True)
    cpu = pltpu.CompilerParams(needs_layout_passes=False,
                               use_tc_tiling_on_sc=False)

    untile = pl.kernel(
        _untile_kernel,
        out_type=(jax.ShapeDtypeStruct((K * NROWS,), jnp.float32),
                  jax.ShapeDtypeStruct((K * NROWS,), jnp.float32)),
        mesh=mesh,
        scratch_types=[
            pltpu.VMEM((T_BLK, 8, 128), jnp.float32),
            pltpu.VMEM((K, TAIL), jnp.float32),
            pltpu.SemaphoreType.DMA,
        ],
        compiler_params=cpt,
    )
    uflat, iflat = untile(user_emb.T, item_emb.T,
                          user_emb[MAIN:].T, item_emb[MAIN:].T)

    gather = pl.kernel(
        _gather_kernel,
        out_type=jax.ShapeDtypeStruct((NW, B_PER_W), jnp.float32),
        mesh=mesh,
        scratch_types=[
            pltpu.VMEM((NQ, 128), jnp.int32),
            pltpu.VMEM((NQ, 128), jnp.int32),
            pltpu.VMEM((K, NQ, 128), jnp.int32),
            pltpu.VMEM((K, NQ, 128), jnp.int32),
            pltpu.VMEM((K, B_PER_W), jnp.float32),
            pltpu.VMEM((K, B_PER_W), jnp.float32),
            pltpu.VMEM((B_PER_W,), jnp.float32),
            pltpu.SemaphoreType.DMA,
        ],
        compiler_params=cpu,
    )
    out = gather(uidx, iidx, uflat, iflat)
    return out.reshape(BATCH)


# speedup vs baseline: 11.0993x; 1.0045x over previous
import jax
import jax.numpy as jnp
from jax import lax
from jax.experimental import pallas as pl
from jax.experimental.pallas import tpu as pltpu
from jax.experimental.pallas import tpu_sc as plsc

NUM_CORES = 2
NUM_SUBCORES = 16
NW = NUM_CORES * NUM_SUBCORES
LANES = 16

BATCH = 16384
K = 32
NROWS = 1000000
B_PER_W = BATCH // NW
NQ = B_PER_W // 128

MAIN = (NROWS // 128) * 128
T_PER_W = (MAIN // 128) // 4
T_BLK = 7
N_IT = T_PER_W // T_BLK
TAIL = NROWS - MAIN


def _untile_kernel(uemb_hbm, iemb_hbm, utail_hbm, itail_hbm,
                   uflat_hbm, iflat_hbm, buf_v, tail_v, sem):
    wid = lax.axis_index("s") * NUM_CORES + lax.axis_index("c")
    t = wid // 16
    r = wid % 16
    g = r // 4
    qd = r % 4
    soff = pl.multiple_of(8 * g, 8)

    def move(emb, flat):
        @pl.loop(0, N_IT)
        def _(it):
            tile0 = qd * T_PER_W + it * T_BLK
            ins = []
            for c in range(T_BLK):
                la = pl.multiple_of((tile0 + c) * 128, 128)
                ins.append(pltpu.async_copy(
                    emb.at[pl.ds(soff, 8), pl.ds(la, 128)],
                    buf_v.at[c], sem))
            for i in ins:
                i.wait()
            for c in range(T_BLK):
                for kr in range(8):
                    doff = pl.multiple_of(
                        (8 * g + kr) * NROWS + (tile0 + c) * 128, 8)
                    pltpu.async_copy(
                        buf_v.at[c, kr],
                        flat.at[pl.ds(doff, 128)], sem)
            for c in range(T_BLK):
                pltpu.make_async_copy(
                    emb.at[pl.ds(soff, 8), pl.ds(0, 128)],
                    buf_v.at[c], sem).wait()

    @pl.when(t == 0)
    def _():
        move(uemb_hbm, uflat_hbm)

    @pl.when(t == 1)
    def _():
        move(iemb_hbm, iflat_hbm)

    def move_tail(tail_hbm, flat):
        pltpu.sync_copy(tail_hbm, tail_v)
        for k in range(K):
            pltpu.async_copy(tail_v.at[k],
                             flat.at[pl.ds(k * NROWS + MAIN, TAIL)],
                             sem).wait()

    @pl.when(wid == 0)
    def _():
        move_tail(utail_hbm, uflat_hbm)

    @pl.when(wid == 16)
    def _():
        move_tail(itail_hbm, iflat_hbm)


def _gather_kernel(uidx_hbm, iidx_hbm, uflat_hbm, iflat_hbm, out_hbm,
                   uidx_v, iidx_v, gidx_u, gidx_i, u_t, v_t, out_v, sem):
    wid = lax.axis_index("s") * NUM_CORES + lax.axis_index("c")

    pltpu.sync_copy(uidx_hbm.at[wid], uidx_v)
    pltpu.sync_copy(iidx_hbm.at[wid], iidx_v)

    @pl.loop(0, NQ)
    def _(q):
        @pl.loop(0, 128 // LANES)
        def _(j):
            ds = pl.ds(j * LANES, LANES)
            ivu = uidx_v[q, ds]
            ivi = iidx_v[q, ds]
            for k in range(K):
                gidx_u[k, q, ds] = ivu + k * NROWS
                gidx_i[k, q, ds] = ivi + k * NROWS

    @pl.loop(0, NQ)
    def _(q):
        cols = pl.ds(q * 128, 128)
        for k in range(K):
            pltpu.async_copy(uflat_hbm.at[gidx_u.at[k, q]],
                             u_t.at[k, cols], sem)
            pltpu.async_copy(iflat_hbm.at[gidx_i.at[k, q]],
                             v_t.at[k, cols], sem)

    @pl.loop(0, NQ)
    def _(q):
        cols = pl.ds(q * 128, 128)
        for k in range(K):
            pltpu.make_async_copy(uflat_hbm.at[gidx_u.at[k, q]],
                                  u_t.at[k, cols], sem).wait()
            pltpu.make_async_copy(iflat_hbm.at[gidx_i.at[k, q]],
                                  v_t.at[k, cols], sem).wait()

    zero = jnp.zeros((LANES,), jnp.float32)

    @pl.loop(0, B_PER_W // LANES)
    def _(c):
        cols = pl.ds(c * LANES, LANES)
        acc = zero
        for k in range(K):
            u = jnp.maximum(u_t[k, cols], zero)
            v = jnp.maximum(v_t[k, cols], zero)
            acc = acc + u * v
        out_v[cols] = acc

    pltpu.sync_copy(out_v, out_hbm.at[wid])


@jax.jit
def kernel(user_idx, item_idx, user_emb, item_emb):
    uidx = user_idx.reshape(NW, NQ, 128)
    iidx = item_idx.reshape(NW, NQ, 128)
    mesh = plsc.VectorSubcoreMesh(core_axis_name="c", subcore_axis_name="s")
    cpt = pltpu.CompilerParams(needs_layout_passes=False,
                               use_tc_tiling_on_sc=True)
    cpu = pltpu.CompilerParams(needs_layout_passes=False,
                               use_tc_tiling_on_sc=False)

    untile = pl.kernel(
        _untile_kernel,
        out_type=(jax.ShapeDtypeStruct((K * NROWS,), jnp.float32),
                  jax.ShapeDtypeStruct((K * NROWS,), jnp.float32)),
        mesh=mesh,
        scratch_types=[
            pltpu.VMEM((T_BLK, 8, 128), jnp.float32),
            pltpu.VMEM((K, TAIL), jnp.float32),
            pltpu.SemaphoreType.DMA,
        ],
        compiler_params=cpt,
    )
    uflat, iflat = untile(user_emb.T, item_emb.T,
                          user_emb[MAIN:].T, item_emb[MAIN:].T)

    gather = pl.kernel(
        _gather_kernel,
        out_type=jax.ShapeDtypeStruct((NW, B_PER_W), jnp.float32),
        mesh=mesh,
        scratch_types=[
            pltpu.VMEM((NQ, 128), jnp.int32),
            pltpu.VMEM((NQ, 128), jnp.int32),
            pltpu.VMEM((K, NQ, 128), jnp.int32),
            pltpu.VMEM((K, NQ, 128), jnp.int32),
            pltpu.VMEM((K, B_PER_W), jnp.float32),
            pltpu.VMEM((K, B_PER_W), jnp.float32),
            pltpu.VMEM((B_PER_W,), jnp.float32),
            pltpu.SemaphoreType.DMA,
        ],
        compiler_params=cpu,
    )
    out = gather(uidx, iidx, uflat, iflat)
    return out.reshape(BATCH)


# speedup vs baseline: 17.2067x; 1.5503x over previous
import jax
import jax.numpy as jnp
from jax import lax
from jax.experimental import pallas as pl
from jax.experimental.pallas import tpu as pltpu
from jax.experimental.pallas import tpu_sc as plsc

NUM_CORES = 2
NUM_SUBCORES = 16
NW = NUM_CORES * NUM_SUBCORES
LANES = 16

BATCH = 16384
K = 32
NROWS = 1000000
B_PER_W = BATCH // NW
NQ = B_PER_W // 128

MAIN = (NROWS // 128) * 128
T_PER_W = (MAIN // 128) // 4
T_BLK = 7
N_IT = T_PER_W // T_BLK
TAIL = NROWS - MAIN


def _untile_kernel(uemb_hbm, iemb_hbm, utail_hbm, itail_hbm,
                   uflat_hbm, iflat_hbm, buf_v, tail_v, sem, sem_in):
    wid = lax.axis_index("s") * NUM_CORES + lax.axis_index("c")
    t = wid // 16
    r = wid % 16
    g = r // 4
    qd = r % 4
    soff = pl.multiple_of(8 * g, 8)

    def move(emb, flat):
        def fire_ins(it, p):
            tile0 = qd * T_PER_W + it * T_BLK
            for c in range(T_BLK):
                la = pl.multiple_of((tile0 + c) * 128, 128)
                pltpu.async_copy(emb.at[pl.ds(soff, 8), pl.ds(la, 128)],
                                 buf_v.at[p, c], sem_in)

        def drain(n_tiles, sem_):
            for c in range(n_tiles):
                pltpu.make_async_copy(emb.at[pl.ds(soff, 8), pl.ds(0, 128)],
                                      buf_v.at[0, c], sem_).wait()

        fire_ins(0, 0)

        @pl.loop(0, N_IT)
        def _(it):
            p = it % 2

            @pl.when(it > 0)
            def _():
                drain(T_BLK, sem)

            @pl.when(it + 1 < N_IT)
            def _():
                fire_ins(it + 1, 1 - p)

            drain(T_BLK, sem_in)
            tile0 = qd * T_PER_W + it * T_BLK
            for c in range(T_BLK):
                for kr in range(8):
                    doff = pl.multiple_of(
                        (8 * g + kr) * NROWS + (tile0 + c) * 128, 8)
                    pltpu.async_copy(
                        buf_v.at[p, c, kr],
                        flat.at[pl.ds(doff, 128)], sem)

        drain(T_BLK, sem)

    @pl.when(t == 0)
    def _():
        move(uemb_hbm, uflat_hbm)

    @pl.when(t == 1)
    def _():
        move(iemb_hbm, iflat_hbm)

    def move_tail(tail_hbm, flat):
        pltpu.sync_copy(tail_hbm, tail_v)
        for k in range(K):
            pltpu.async_copy(tail_v.at[k],
                             flat.at[pl.ds(k * NROWS + MAIN, TAIL)],
                             sem).wait()

    @pl.when(wid == 0)
    def _():
        move_tail(utail_hbm, uflat_hbm)

    @pl.when(wid == 16)
    def _():
        move_tail(itail_hbm, iflat_hbm)


def _gather_kernel(uidx_hbm, iidx_hbm, uflat_hbm, iflat_hbm, out_hbm,
                   uidx_v, iidx_v, gidx_u, gidx_i, u_t, v_t, out_v, sem):
    wid = lax.axis_index("s") * NUM_CORES + lax.axis_index("c")

    pltpu.sync_copy(uidx_hbm.at[wid], uidx_v)
    pltpu.sync_copy(iidx_hbm.at[wid], iidx_v)

    @pl.loop(0, NQ)
    def _(q):
        @pl.loop(0, 128 // LANES)
        def _(j):
            ds = pl.ds(j * LANES, LANES)
            ivu = uidx_v[q, ds]
            ivi = iidx_v[q, ds]
            for k in range(K):
                gidx_u[k, q, ds] = ivu + k * NROWS
                gidx_i[k, q, ds] = ivi + k * NROWS

    @pl.loop(0, NQ)
    def _(q):
        cols = pl.ds(q * 128, 128)
        for k in range(K):
            pltpu.async_copy(uflat_hbm.at[gidx_u.at[k, q]],
                             u_t.at[k, cols], sem)
            pltpu.async_copy(iflat_hbm.at[gidx_i.at[k, q]],
                             v_t.at[k, cols], sem)

    @pl.loop(0, NQ)
    def _(q):
        cols = pl.ds(q * 128, 128)
        for k in range(K):
            pltpu.make_async_copy(uflat_hbm.at[gidx_u.at[k, q]],
                                  u_t.at[k, cols], sem).wait()
            pltpu.make_async_copy(iflat_hbm.at[gidx_i.at[k, q]],
                                  v_t.at[k, cols], sem).wait()

    zero = jnp.zeros((LANES,), jnp.float32)

    @pl.loop(0, B_PER_W // LANES)
    def _(c):
        cols = pl.ds(c * LANES, LANES)
        acc = zero
        for k in range(K):
            u = jnp.maximum(u_t[k, cols], zero)
            v = jnp.maximum(v_t[k, cols], zero)
            acc = acc + u * v
        out_v[cols] = acc

    pltpu.sync_copy(out_v, out_hbm.at[wid])


@jax.jit
def kernel(user_idx, item_idx, user_emb, item_emb):
    uidx = user_idx.reshape(NW, NQ, 128)
    iidx = item_idx.reshape(NW, NQ, 128)
    mesh = plsc.VectorSubcoreMesh(core_axis_name="c", subcore_axis_name="s")
    cpt = pltpu.CompilerParams(needs_layout_passes=False,
                               use_tc_tiling_on_sc=True)
    cpu = pltpu.CompilerParams(needs_layout_passes=False,
                               use_tc_tiling_on_sc=False)

    untile = pl.kernel(
        _untile_kernel,
        out_type=(jax.ShapeDtypeStruct((K * NROWS,), jnp.float32),
                  jax.ShapeDtypeStruct((K * NROWS,), jnp.float32)),
        mesh=mesh,
        scratch_types=[
            pltpu.VMEM((2, T_BLK, 8, 128), jnp.float32),
            pltpu.VMEM((K, TAIL), jnp.float32),
            pltpu.SemaphoreType.DMA,
            pltpu.SemaphoreType.DMA,
        ],
        compiler_params=cpt,
    )
    uflat, iflat = untile(user_emb.T, item_emb.T,
                          user_emb[MAIN:].T, item_emb[MAIN:].T)

    gather = pl.kernel(
        _gather_kernel,
        out_type=jax.ShapeDtypeStruct((NW, B_PER_W), jnp.float32),
        mesh=mesh,
        scratch_types=[
            pltpu.VMEM((NQ, 128), jnp.int32),
            pltpu.VMEM((NQ, 128), jnp.int32),
            pltpu.VMEM((K, NQ, 128), jnp.int32),
            pltpu.VMEM((K, NQ, 128), jnp.int32),
            pltpu.VMEM((K, B_PER_W), jnp.float32),
            pltpu.VMEM((K, B_PER_W), jnp.float32),
            pltpu.VMEM((B_PER_W,), jnp.float32),
            pltpu.SemaphoreType.DMA,
        ],
        compiler_params=cpu,
    )
    out = gather(uidx, iidx, uflat, iflat)
    return out.reshape(BATCH)


# speedup vs baseline: 18.2636x; 1.0614x over previous
import jax
import jax.numpy as jnp
from jax import lax
from jax.experimental import pallas as pl
from jax.experimental.pallas import tpu as pltpu
from jax.experimental.pallas import tpu_sc as plsc

NUM_CORES = 2
NUM_SUBCORES = 16
NW = NUM_CORES * NUM_SUBCORES
LANES = 16

BATCH = 16384
K = 32
NROWS = 1000000
B_PER_W = BATCH // NW
NQ = B_PER_W // 128

MAIN = (NROWS // 128) * 128
T_PER_W = (MAIN // 128) // 4
T_BLK = 7
N_IT = T_PER_W // T_BLK
TAIL = NROWS - MAIN


def _untile_kernel(uemb_hbm, iemb_hbm, utail_hbm, itail_hbm,
                   uflat_hbm, iflat_hbm, buf_v, tail_v, sem, sem_in):
    wid = lax.axis_index("s") * NUM_CORES + lax.axis_index("c")
    t = wid // 16
    r = wid % 16
    g = r // 4
    qd = r % 4
    soff = pl.multiple_of(8 * g, 8)

    def move(emb, flat):
        def fire_ins(it, p):
            tile0 = qd * T_PER_W + it * T_BLK
            for c in range(T_BLK):
                la = pl.multiple_of((tile0 + c) * 128, 128)
                pltpu.async_copy(emb.at[pl.ds(soff, 8), pl.ds(la, 128)],
                                 buf_v.at[p, c], sem_in)

        def drain(n_tiles, sem_):
            for c in range(n_tiles):
                pltpu.make_async_copy(emb.at[pl.ds(soff, 8), pl.ds(0, 128)],
                                      buf_v.at[0, c], sem_).wait()

        fire_ins(0, 0)

        @pl.loop(0, N_IT)
        def _(it):
            p = it % 3

            @pl.when(it > 1)
            def _():
                drain(T_BLK, sem)

            @pl.when(it + 1 < N_IT)
            def _():
                fire_ins(it + 1, (it + 1) % 3)

            drain(T_BLK, sem_in)
            tile0 = qd * T_PER_W + it * T_BLK
            for c in range(T_BLK):
                for kr in range(8):
                    doff = pl.multiple_of(
                        (8 * g + kr) * NROWS + (tile0 + c) * 128, 8)
                    pltpu.async_copy(
                        buf_v.at[p, c, kr],
                        flat.at[pl.ds(doff, 128)], sem)

        drain(T_BLK, sem)
        drain(T_BLK, sem)

    @pl.when(t == 0)
    def _():
        move(uemb_hbm, uflat_hbm)

    @pl.when(t == 1)
    def _():
        move(iemb_hbm, iflat_hbm)

    def move_tail(tail_hbm, flat):
        pltpu.sync_copy(tail_hbm, tail_v)
        for k in range(K):
            pltpu.async_copy(tail_v.at[k],
                             flat.at[pl.ds(k * NROWS + MAIN, TAIL)],
                             sem).wait()

    @pl.when(wid == 0)
    def _():
        move_tail(utail_hbm, uflat_hbm)

    @pl.when(wid == 16)
    def _():
        move_tail(itail_hbm, iflat_hbm)


def _gather_kernel(uidx_hbm, iidx_hbm, uflat_hbm, iflat_hbm, out_hbm,
                   uidx_v, iidx_v, gidx_u, gidx_i, u_t, v_t, out_v, sem):
    wid = lax.axis_index("s") * NUM_CORES + lax.axis_index("c")

    pltpu.sync_copy(uidx_hbm.at[wid], uidx_v)
    pltpu.sync_copy(iidx_hbm.at[wid], iidx_v)

    @pl.loop(0, NQ)
    def _(q):
        @pl.loop(0, 128 // LANES)
        def _(j):
            ds = pl.ds(j * LANES, LANES)
            ivu = uidx_v[q, ds]
            ivi = iidx_v[q, ds]
            for k in range(K):
                gidx_u[k, q, ds] = ivu + k * NROWS
                gidx_i[k, q, ds] = ivi + k * NROWS

    @pl.loop(0, NQ)
    def _(q):
        cols = pl.ds(q * 128, 128)
        for k in range(K):
            pltpu.async_copy(uflat_hbm.at[gidx_u.at[k, q]],
                             u_t.at[k, cols], sem)
            pltpu.async_copy(iflat_hbm.at[gidx_i.at[k, q]],
                             v_t.at[k, cols], sem)

    @pl.loop(0, NQ)
    def _(q):
        cols = pl.ds(q * 128, 128)
        for k in range(K):
            pltpu.make_async_copy(uflat_hbm.at[gidx_u.at[k, q]],
                                  u_t.at[k, cols], sem).wait()
            pltpu.make_async_copy(iflat_hbm.at[gidx_i.at[k, q]],
                                  v_t.at[k, cols], sem).wait()

    zero = jnp.zeros((LANES,), jnp.float32)

    @pl.loop(0, B_PER_W // LANES)
    def _(c):
        cols = pl.ds(c * LANES, LANES)
        acc = zero
        for k in range(K):
            u = jnp.maximum(u_t[k, cols], zero)
            v = jnp.maximum(v_t[k, cols], zero)
            acc = acc + u * v
        out_v[cols] = acc

    pltpu.sync_copy(out_v, out_hbm.at[wid])


@jax.jit
def kernel(user_idx, item_idx, user_emb, item_emb):
    uidx = user_idx.reshape(NW, NQ, 128)
    iidx = item_idx.reshape(NW, NQ, 128)
    mesh = plsc.VectorSubcoreMesh(core_axis_name="c", subcore_axis_name="s")
    cpt = pltpu.CompilerParams(needs_layout_passes=False,
                               use_tc_tiling_on_sc=True)
    cpu = pltpu.CompilerParams(needs_layout_passes=False,
                               use_tc_tiling_on_sc=False)

    untile = pl.kernel(
        _untile_kernel,
        out_type=(jax.ShapeDtypeStruct((K * NROWS,), jnp.float32),
                  jax.ShapeDtypeStruct((K * NROWS,), jnp.float32)),
        mesh=mesh,
        scratch_types=[
            pltpu.VMEM((3, T_BLK, 8, 128), jnp.float32),
            pltpu.VMEM((K, TAIL), jnp.float32),
            pltpu.SemaphoreType.DMA,
            pltpu.SemaphoreType.DMA,
        ],
        compiler_params=cpt,
    )
    uflat, iflat = untile(user_emb.T, item_emb.T,
                          user_emb[MAIN:].T, item_emb[MAIN:].T)

    gather = pl.kernel(
        _gather_kernel,
        out_type=jax.ShapeDtypeStruct((NW, B_PER_W), jnp.float32),
        mesh=mesh,
        scratch_types=[
            pltpu.VMEM((NQ, 128), jnp.int32),
            pltpu.VMEM((NQ, 128), jnp.int32),
            pltpu.VMEM((K, NQ, 128), jnp.int32),
            pltpu.VMEM((K, NQ, 128), jnp.int32),
            pltpu.VMEM((K, B_PER_W), jnp.float32),
            pltpu.VMEM((K, B_PER_W), jnp.float32),
            pltpu.VMEM((B_PER_W,), jnp.float32),
            pltpu.SemaphoreType.DMA,
        ],
        compiler_params=cpu,
    )
    out = gather(uidx, iidx, uflat, iflat)
    return out.reshape(BATCH)
